# Initial kernel scaffold; baseline (speedup 1.0000x reference)
import jax
import jax.numpy as jnp
from jax.experimental import pallas as pl


def kernel(x, edge_index, edge_attr, edge_vec, edge_length, We1, be1, We2, be2, Wn1, bn1, Wn2, bn2, Wt1, bt1, Wt2, bt2):
    raise NotImplementedError("write your pallas kernel here")


# speedup vs baseline: 2.8106x; 2.8106x over previous
import functools

import jax
import jax.numpy as jnp
from jax import lax
from jax.experimental import pallas as pl
from jax.experimental.pallas import tpu as pltpu
from jax.experimental.pallas import tpu_sc as plsc

F32 = jnp.float32

_N = 10000
_E = 320000
_D = 128

_NC = 2
_NS = 16
_NW = _NC * _NS
_G = 80
_EPW = _E // _NW
_NG1 = _EPW // _G
_EPH = _E // _NC
_NP = 10240
_NRT = _NP // _NS
_ZR = 128


def _swish(v):
    return v * jax.nn.sigmoid(v)


def _precomp_body(x_ref, wa_ref, wc_ref, xa_ref, xc_ref):
    xv = x_ref[...]
    xa_ref[...] = jnp.dot(xv, wa_ref[...], preferred_element_type=F32)
    xc_ref[...] = jnp.dot(xv, wc_ref[...], preferred_element_type=F32)


def _precomp(x, wa, wc):
    bn = 2000
    grid = (_N // bn,)
    return pl.pallas_call(
        _precomp_body,
        grid=grid,
        in_specs=[
            pl.BlockSpec((bn, _D), lambda i: (i, 0)),
            pl.BlockSpec((_D, _D), lambda i: (0, 0)),
            pl.BlockSpec((_D, _D), lambda i: (0, 0)),
        ],
        out_specs=[
            pl.BlockSpec((bn, _D), lambda i: (i, 0)),
            pl.BlockSpec((bn, _D), lambda i: (i, 0)),
        ],
        out_shape=[
            jax.ShapeDtypeStruct((_N, _D), F32),
            jax.ShapeDtypeStruct((_N, _D), F32),
        ],
    )(x, wa, wc)


def _sc_vecstats_body(evp, col, znaux, out, cidx, evbuf, wbuf, naux_sh, sem0):
    cid = lax.axis_index("c")
    sid = lax.axis_index("s")

    def zw(r, _):
        for k in range(_D // 16):
            wbuf[r, pl.ds(k * 16, 16)] = jnp.zeros((16,), F32)
        return 0

    lax.fori_loop(0, _G, zw, 0)

    pltpu.sync_copy(znaux.at[pl.ds(sid * _NRT, _NRT)],
                    naux_sh.at[pl.ds(sid * _NRT, _NRT)])
    plsc.subcore_barrier()

    def g(gi, _):
        off = cid * _EPH + sid * _EPW + gi * _G
        pltpu.sync_copy(col.at[pl.ds(off, _G)], cidx)
        pltpu.sync_copy(evp.at[pl.ds(off, _G)], evbuf)

        def expand(r, _):
            wbuf[r, pl.ds(0, 16)] = evbuf[r, :]
            return 0

        lax.fori_loop(0, _G, expand, 0)
        pltpu.sync_copy(wbuf, naux_sh.at[cidx], add=True)
        return 0

    lax.fori_loop(0, _NG1, g, 0)
    plsc.subcore_barrier()

    pltpu.sync_copy(naux_sh.at[pl.ds(sid * _NRT, _NRT)],
                    out.at[cid, pl.ds(sid * _NRT, _NRT)])


def _sc_vecstats(evp, col):
    mesh = plsc.VectorSubcoreMesh(core_axis_name="c", subcore_axis_name="s")
    f = functools.partial(
        pl.kernel,
        out_type=jax.ShapeDtypeStruct((_NC, _NP, _D), F32),
        mesh=mesh,
        scratch_types=[
            pltpu.VMEM((_G,), jnp.int32),
            pltpu.VMEM((_G, 16), F32),
            pltpu.VMEM((_G, _D), F32),
            pltpu.VMEM_SHARED((_NP, _D), F32),
            pltpu.SemaphoreType.DMA,
        ],
    )(_sc_vecstats_body)
    return f(evp, col, jnp.zeros((_NP, _D), F32))


def _combine_body(a0, a1, out):
    out[...] = a0[0] + a1[0]


def _combine(naux2):
    bn = 2048
    grid = (_NP // bn,)
    return pl.pallas_call(
        _combine_body,
        grid=grid,
        in_specs=[
            pl.BlockSpec((1, bn, _D), lambda i: (0, i, 0)),
            pl.BlockSpec((1, bn, _D), lambda i: (1, i, 0)),
        ],
        out_specs=pl.BlockSpec((bn, _D), lambda i: (i, 0)),
        out_shape=jax.ShapeDtypeStruct((_NP, _D), F32),
    )(naux2, naux2)


def _sc_gather_body(xa, xc, naux, row, col, gath, ctx,
                    ridx, cidx, abuf, bbuf, sbuf, cbuf, sem0, sem1, sem2):
    cid = lax.axis_index("c")
    sid = lax.axis_index("s")
    wid = sid * _NC + cid

    def g1(g, _):
        off = wid * _EPW + g * _G
        pltpu.sync_copy(row.at[pl.ds(off, _G)], ridx)
        pltpu.sync_copy(col.at[pl.ds(off, _G)], cidx)
        cp_a = pltpu.async_copy(xa.at[ridx], abuf, sem0)
        cp_b = pltpu.async_copy(xc.at[cidx], bbuf, sem1)
        cp_s = pltpu.async_copy(naux.at[cidx], sbuf, sem2)
        cp_a.wait()
        cp_b.wait()

        def add_row(r, _):
            for k in range(_D // 16):
                s = pl.ds(k * 16, 16)
                abuf[r, s] = abuf[r, s] + bbuf[r, s]
            return 0

        lax.fori_loop(0, _G, add_row, 0)
        pltpu.sync_copy(abuf, gath.at[pl.ds(off, _G)])
        cp_s.wait()

        def compact(r, _):
            cbuf[r, :] = sbuf[r, pl.ds(0, 16)]
            return 0

        lax.fori_loop(0, _G, compact, 0)
        pltpu.sync_copy(cbuf, ctx.at[pl.ds(off, _G)])
        return 0

    lax.fori_loop(0, _NG1, g1, 0)


def _sc_gather(xa, xc, naux, row, col):
    mesh = plsc.VectorSubcoreMesh(core_axis_name="c", subcore_axis_name="s")
    f = functools.partial(
        pl.kernel,
        out_type=[
            jax.ShapeDtypeStruct((_E, _D), F32),
            jax.ShapeDtypeStruct((_E, 16), F32),
        ],
        mesh=mesh,
        scratch_types=[
            pltpu.VMEM((_G,), jnp.int32),
            pltpu.VMEM((_G,), jnp.int32),
            pltpu.VMEM((_G, _D), F32),
            pltpu.VMEM((_G, _D), F32),
            pltpu.VMEM((_G, _D), F32),
            pltpu.VMEM((_G, 16), F32),
            pltpu.SemaphoreType.DMA,
            pltpu.SemaphoreType.DMA,
            pltpu.SemaphoreType.DMA,
        ],
    )(_sc_gather_body)
    return f(xa, xc, naux, row, col)


def _edge_body(gath, ea, ctx, evp, w1b, b1, w2, b2, wt1, wtc, bt1, wt2, bt2,
               out):
    lane = lax.broadcasted_iota(jnp.int32, (1, 16), 1)
    m3 = (lane < 3).astype(F32)
    ctxv = ctx[...]
    evpv = evp[...]
    dot_raw = jnp.sum(ctxv * evpv * m3, axis=1, keepdims=True)
    n2_raw = jnp.sum(ctxv * ctxv * m3, axis=1, keepdims=True)
    cnt = jnp.sum(ctxv * (lane == 3), axis=1, keepdims=True)
    el = jnp.sum(evpv * (lane == 4), axis=1, keepdims=True)
    c = jnp.maximum(cnt, 1.0)
    cos = jnp.clip(dot_raw / (el * (jnp.sqrt(n2_raw) + 1e-6 * c)), -1.0, 1.0)

    h1 = _swish(gath[...] + b1[...]
                + jnp.dot(ea[...], w1b[...], preferred_element_type=F32))
    m = _swish(jnp.dot(h1, w2[...], preferred_element_type=F32) + b2[...])
    t = _swish(jnp.dot(m, wt1[...], preferred_element_type=F32)
               + cos * wtc[...] + bt1[...])
    g = jax.nn.sigmoid(jnp.dot(t, wt2[...], preferred_element_type=F32)
                       + bt2[...])
    out[...] = m * g


def _edge_mlp(gath, ea, ctx, evp, w1b, b1, w2, b2, wt1, wtc, bt1, wt2, bt2):
    be = 2000
    grid = (_E // be,)
    wspec = pl.BlockSpec((_D, _D), lambda i: (0, 0))
    bspec = pl.BlockSpec((1, _D), lambda i: (0, 0))
    return pl.pallas_call(
        _edge_body,
        grid=grid,
        in_specs=[
            pl.BlockSpec((be, _D), lambda i: (i, 0)),
            pl.BlockSpec((be, 16), lambda i: (i, 0)),
            pl.BlockSpec((be, 16), lambda i: (i, 0)),
            pl.BlockSpec((be, 16), lambda i: (i, 0)),
            pl.BlockSpec((16, _D), lambda i: (0, 0)),
            bspec, wspec, bspec, wspec, bspec, bspec, wspec, bspec,
        ],
        out_specs=pl.BlockSpec((be, _D), lambda i: (i, 0)),
        out_shape=jax.ShapeDtypeStruct((_E, _D), F32),
    )(gath, ea, ctx, evp, w1b, b1, w2, b2, wt1, wtc, bt1, wt2, bt2)


def _sc_scatter_body(mp, col, out, cidx, mbuf, zbuf, aggr_sh):
    cid = lax.axis_index("c")
    sid = lax.axis_index("s")

    def zrow(r, _):
        for k in range(_D // 16):
            zbuf[r, pl.ds(k * 16, 16)] = jnp.zeros((16,), F32)
        return 0

    lax.fori_loop(0, _ZR, zrow, 0)
    for q in range(_NRT // _ZR):
        pltpu.sync_copy(zbuf, aggr_sh.at[pl.ds(sid * _NRT + q * _ZR, _ZR)])
    plsc.subcore_barrier()

    def g(gi, _):
        off = cid * _EPH + sid * _EPW + gi * _G
        pltpu.sync_copy(col.at[pl.ds(off, _G)], cidx)
        pltpu.sync_copy(mp.at[pl.ds(off, _G)], mbuf)
        pltpu.sync_copy(mbuf, aggr_sh.at[cidx], add=True)
        return 0

    lax.fori_loop(0, _NG1, g, 0)
    plsc.subcore_barrier()

    for q in range(_NRT // _ZR):
        r0 = sid * _NRT + q * _ZR
        pltpu.sync_copy(aggr_sh.at[pl.ds(r0, _ZR)], out.at[cid, pl.ds(r0, _ZR)])


def _sc_scatter(mp, col):
    mesh = plsc.VectorSubcoreMesh(core_axis_name="c", subcore_axis_name="s")
    f = functools.partial(
        pl.kernel,
        out_type=jax.ShapeDtypeStruct((_NC, _NP, _D), F32),
        mesh=mesh,
        scratch_types=[
            pltpu.VMEM((_G,), jnp.int32),
            pltpu.VMEM((_G, _D), F32),
            pltpu.VMEM((_ZR, _D), F32),
            pltpu.VMEM_SHARED((_NP, _D), F32),
        ],
    )(_sc_scatter_body)
    return f(mp, col)


def _node_body(x_ref, a0, a1, wna, wnb, b1, w2, b2, out):
    xv = x_ref[...]
    aggr = a0[0] + a1[0]
    h = _swish(jnp.dot(xv, wna[...], preferred_element_type=F32)
               + jnp.dot(aggr, wnb[...], preferred_element_type=F32)
               + b1[...])
    out[...] = xv + _swish(jnp.dot(h, w2[...], preferred_element_type=F32)
                           + b2[...])


def _node_mlp(x, aggr2, wna, wnb, b1, w2, b2):
    bn = 2000
    grid = (_N // bn,)
    wspec = pl.BlockSpec((_D, _D), lambda i: (0, 0))
    bspec = pl.BlockSpec((1, _D), lambda i: (0, 0))
    return pl.pallas_call(
        _node_body,
        grid=grid,
        in_specs=[
            pl.BlockSpec((bn, _D), lambda i: (i, 0)),
            pl.BlockSpec((1, bn, _D), lambda i: (0, i, 0)),
            pl.BlockSpec((1, bn, _D), lambda i: (1, i, 0)),
            wspec, wspec, bspec, wspec, bspec,
        ],
        out_specs=pl.BlockSpec((bn, _D), lambda i: (i, 0)),
        out_shape=jax.ShapeDtypeStruct((_N, _D), F32),
    )(x, aggr2, aggr2, wna, wnb, b1, w2, b2)


def kernel(x, edge_index, edge_attr, edge_vec, edge_length,
           We1, be1, We2, be2, Wn1, bn1, Wn2, bn2, Wt1, bt1, Wt2, bt2):
    D = _D
    R = edge_attr.shape[1]
    row = edge_index[0]
    col = edge_index[1]

    We1a, We1b, We1c = We1[:D], We1[D:D + R], We1[D + R:]
    evp = jnp.concatenate(
        [edge_vec,
         jnp.ones((_E, 1), F32),
         edge_length[:, None],
         jnp.zeros((_E, 11), F32)], axis=1)

    xa, xc = _precomp(x, We1a, We1c)
    naux2 = _sc_vecstats(evp, col)
    naux = _combine(naux2)
    gath, ctx = _sc_gather(xa, xc, naux, row, col)
    mp = _edge_mlp(gath, edge_attr, ctx, evp,
                   We1b, be1[None, :], We2, be2[None, :],
                   Wt1[:D], Wt1[D:D + 1], bt1[None, :], Wt2, bt2[None, :])
    aggr2 = _sc_scatter(mp, col)
    return _node_mlp(x, aggr2, Wn1[:D], Wn1[D:], bn1[None, :], Wn2,
                     bn2[None, :])


# speedup vs baseline: 4.4709x; 1.5908x over previous
import functools

import jax
import jax.numpy as jnp
from jax import lax
from jax.experimental import pallas as pl
from jax.experimental.pallas import tpu as pltpu
from jax.experimental.pallas import tpu_sc as plsc

F32 = jnp.float32

_N = 10000
_E = 320000
_D = 128

_NC = 2
_NS = 16
_NW = _NC * _NS
_G = 80
_EPW = _E // _NW
_NG1 = _EPW // _G
_EPH = _E // _NC
_NP = 10240
_NRT = _NP // _NS
_ZR = 128
_CK = 5
_GPC = _NG1 // _CK
_EPC = _E // _CK


def _swish(v):
    return v * jax.nn.sigmoid(v)


def _precomp_body(x_ref, wa_ref, wc_ref, xa_ref, xc_ref):
    xv = x_ref[...]
    xa_ref[...] = jnp.dot(xv, wa_ref[...], preferred_element_type=F32)
    xc_ref[...] = jnp.dot(xv, wc_ref[...], preferred_element_type=F32)


def _precomp(x, wa, wc):
    bn = 2000
    grid = (_N // bn,)
    return pl.pallas_call(
        _precomp_body,
        grid=grid,
        in_specs=[
            pl.BlockSpec((bn, _D), lambda i: (i, 0)),
            pl.BlockSpec((_D, _D), lambda i: (0, 0)),
            pl.BlockSpec((_D, _D), lambda i: (0, 0)),
        ],
        out_specs=[
            pl.BlockSpec((bn, _D), lambda i: (i, 0)),
            pl.BlockSpec((bn, _D), lambda i: (i, 0)),
        ],
        out_shape=[
            jax.ShapeDtypeStruct((_N, _D), F32),
            jax.ShapeDtypeStruct((_N, _D), F32),
        ],
    )(x, wa, wc)


def _evp_body(ev, el, out):
    bn = ev.shape[0]
    out[...] = jnp.concatenate(
        [ev[...], jnp.ones((bn, 1), F32), el[...],
         jnp.zeros((bn, 11), F32)], axis=1)


def _build_evp(edge_vec, edge_length):
    bn = 20000
    grid = (_E // bn,)
    return pl.pallas_call(
        _evp_body,
        grid=grid,
        in_specs=[
            pl.BlockSpec((bn, 3), lambda i: (i, 0)),
            pl.BlockSpec((bn, 1), lambda i: (i, 0)),
        ],
        out_specs=pl.BlockSpec((bn, 16), lambda i: (i, 0)),
        out_shape=jax.ShapeDtypeStruct((_E, 16), F32),
    )(edge_vec, edge_length)


def _sc_vecstats_body(evp, col2d, znaux, out, idx0, idx1, ebuf0, ebuf1,
                      wbuf0, wbuf1, naux_sh, esem0, esem1, ssem0, ssem1):
    cid = lax.axis_index("c")
    sid = lax.axis_index("s")
    base = cid * _EPH + sid * _EPW
    chunk = base // _EPW
    idxs = (idx0, idx1)
    ebufs, wbufs = (ebuf0, ebuf1), (wbuf0, wbuf1)
    esems, ssems = (esem0, esem1), (ssem0, ssem1)

    def fire_in(g, b):
        pltpu.async_copy(col2d.at[chunk, g], idxs[b], esems[b])
        pltpu.async_copy(evp.at[pl.ds(base + g * _G, _G)], ebufs[b], esems[b])

    def wait_in(b):
        pltpu.make_async_copy(col2d.at[0, 0], idxs[b], esems[b]).wait()
        pltpu.make_async_copy(evp.at[pl.ds(0, _G)], ebufs[b], esems[b]).wait()

    def wait_scat(b):
        pltpu.make_async_copy(znaux.at[pl.ds(0, _G)], wbufs[b],
                              ssems[b]).wait()

    def zw(r, _):
        for k in range(_D // 16):
            wbuf0[r, pl.ds(k * 16, 16)] = jnp.zeros((16,), F32)
            wbuf1[r, pl.ds(k * 16, 16)] = jnp.zeros((16,), F32)
        return 0

    lax.fori_loop(0, _G, zw, 0)

    pltpu.sync_copy(znaux.at[pl.ds(sid * _NRT, _NRT)],
                    naux_sh.at[pl.ds(sid * _NRT, _NRT)])
    plsc.subcore_barrier()

    fire_in(0, 0)
    fire_in(1, 1)

    def consume(g, b):
        wait_in(b)

        def expand(r, _):
            wbufs[b][r, pl.ds(0, 16)] = ebufs[b][r, :]
            return 0

        lax.fori_loop(0, _G, expand, 0)
        pltpu.async_copy(wbufs[b], naux_sh.at[idxs[b]], ssems[b], add=True)

    consume(0, 0)

    def body(g, b, prefetch=True):
        wait_scat(1 - b)
        if prefetch:
            fire_in(g + 1, 1 - b)
        consume(g, b)

    body(1, 1)

    def pair(p, _):
        body(2 * p, 0)
        body(2 * p + 1, 1)
        return 0

    lax.fori_loop(1, (_NG1 - 1) // 2, pair, 0)
    body(_NG1 - 1, 0, prefetch=False)
    wait_scat(0)
    plsc.subcore_barrier()

    pltpu.sync_copy(naux_sh.at[pl.ds(sid * _NRT, _NRT)],
                    out.at[cid, pl.ds(sid * _NRT, _NRT)])


def _sc_vecstats(evp, col2d):
    mesh = plsc.VectorSubcoreMesh(core_axis_name="c", subcore_axis_name="s")
    f = functools.partial(
        pl.kernel,
        out_type=jax.ShapeDtypeStruct((_NC, _NP, _D), F32),
        mesh=mesh,
        scratch_types=[
            pltpu.VMEM((_G,), jnp.int32),
            pltpu.VMEM((_G,), jnp.int32),
            pltpu.VMEM((_G, 16), F32),
            pltpu.VMEM((_G, 16), F32),
            pltpu.VMEM((_G, _D), F32),
            pltpu.VMEM((_G, _D), F32),
            pltpu.VMEM_SHARED((_NP, _D), F32),
            pltpu.SemaphoreType.DMA,
            pltpu.SemaphoreType.DMA,
            pltpu.SemaphoreType.DMA,
            pltpu.SemaphoreType.DMA,
        ],
    )(_sc_vecstats_body)
    return f(evp, col2d, jnp.zeros((_NP, _D), F32))


def _combine_body(a0, a1, out):
    out[...] = a0[0] + a1[0]


def _combine(naux2):
    bn = 2048
    grid = (_NP // bn,)
    return pl.pallas_call(
        _combine_body,
        grid=grid,
        in_specs=[
            pl.BlockSpec((1, bn, _D), lambda i: (0, i, 0)),
            pl.BlockSpec((1, bn, _D), lambda i: (1, i, 0)),
        ],
        out_specs=pl.BlockSpec((bn, _D), lambda i: (i, 0)),
        out_shape=jax.ShapeDtypeStruct((_NP, _D), F32),
    )(naux2, naux2)


def _sc_gather_body(xa, xc, naux, row2d, col2d, gath, ctx,
                    ridx0, ridx1, cidx0, cidx1, abuf0, abuf1, bbuf0, bbuf1,
                    sbuf0, sbuf1, cbuf0, cbuf1, isem0, isem1, gsem0, gsem1,
                    osem0, osem1):
    cid = lax.axis_index("c")
    sid = lax.axis_index("s")
    wid = sid * _NC + cid
    base = wid * _EPW
    ridxs, cidxs = (ridx0, ridx1), (cidx0, cidx1)
    abufs, bbufs, sbufs, cbufs = ((abuf0, abuf1), (bbuf0, bbuf1),
                                  (sbuf0, sbuf1), (cbuf0, cbuf1))
    isems, gsems, osems = (isem0, isem1), (gsem0, gsem1), (osem0, osem1)

    def fire_idx(g, b):
        pltpu.async_copy(row2d.at[wid, g], ridxs[b], isems[b])
        pltpu.async_copy(col2d.at[wid, g], cidxs[b], isems[b])

    def wait_idx(b):
        pltpu.make_async_copy(row2d.at[0, 0], ridxs[b], isems[b]).wait()
        pltpu.make_async_copy(row2d.at[0, 0], cidxs[b], isems[b]).wait()

    def fire_gathers(b):
        pltpu.async_copy(xa.at[ridxs[b]], abufs[b], gsems[b])
        pltpu.async_copy(xc.at[cidxs[b]], bbufs[b], gsems[b])
        pltpu.async_copy(naux.at[cidxs[b]], sbufs[b], gsems[b])

    def wait_gathers(b):
        for _ in range(3):
            pltpu.make_async_copy(xa.at[pl.ds(0, _G)], abufs[b],
                                  gsems[b]).wait()

    def consume(g, b):
        off = base + g * _G
        wait_gathers(b)

        def add_row(r, _):
            for k in range(_D // 16):
                s = pl.ds(k * 16, 16)
                abufs[b][r, s] = abufs[b][r, s] + bbufs[b][r, s]
            return 0

        lax.fori_loop(0, _G, add_row, 0)

        def compact(r, _):
            cbufs[b][r, :] = sbufs[b][r, pl.ds(0, 16)]
            return 0

        lax.fori_loop(0, _G, compact, 0)
        pltpu.async_copy(abufs[b], gath.at[pl.ds(off, _G)], osems[b])
        pltpu.async_copy(cbufs[b], ctx.at[pl.ds(off, _G)], osems[b])

    def wait_outs(b):
        pltpu.make_async_copy(xa.at[pl.ds(0, _G)], abufs[b], osems[b]).wait()
        pltpu.make_async_copy(ctx.at[pl.ds(0, _G)], cbufs[b], osems[b]).wait()

    fire_idx(0, 0)
    fire_idx(1, 1)
    wait_idx(0)
    fire_gathers(0)
    wait_idx(1)
    fire_gathers(1)
    consume(0, 0)

    def body(g, b, prefetch=True):
        if prefetch:
            fire_idx(g + 1, 1 - b)
        wait_outs(1 - b)
        if prefetch:
            wait_idx(1 - b)
            fire_gathers(1 - b)
        consume(g, b)

    body(1, 1)

    def pair(p, _):
        body(2 * p, 0)
        body(2 * p + 1, 1)
        return 0

    lax.fori_loop(1, (_NG1 - 1) // 2, pair, 0)
    body(_NG1 - 1, 0, prefetch=False)
    wait_outs(0)


def _sc_gather(xa, xc, naux, row2d, col2d):
    mesh = plsc.VectorSubcoreMesh(core_axis_name="c", subcore_axis_name="s")
    f = functools.partial(
        pl.kernel,
        out_type=[
            jax.ShapeDtypeStruct((_E, _D), F32),
            jax.ShapeDtypeStruct((_E, 16), F32),
        ],
        mesh=mesh,
        scratch_types=[
            pltpu.VMEM((_G,), jnp.int32),
            pltpu.VMEM((_G,), jnp.int32),
            pltpu.VMEM((_G,), jnp.int32),
            pltpu.VMEM((_G,), jnp.int32),
            pltpu.VMEM((_G, _D), F32),
            pltpu.VMEM((_G, _D), F32),
            pltpu.VMEM((_G, _D), F32),
            pltpu.VMEM((_G, _D), F32),
            pltpu.VMEM((_G, _D), F32),
            pltpu.VMEM((_G, _D), F32),
            pltpu.VMEM((_G, 16), F32),
            pltpu.VMEM((_G, 16), F32),
            pltpu.SemaphoreType.DMA,
            pltpu.SemaphoreType.DMA,
            pltpu.SemaphoreType.DMA,
            pltpu.SemaphoreType.DMA,
            pltpu.SemaphoreType.DMA,
            pltpu.SemaphoreType.DMA,
        ],
    )(_sc_gather_body)
    return f(xa, xc, naux, row2d, col2d)


def _edge_body(gath, ea, ctx, evp, s3, e3, e4, w1b, b1, w2, b2, wt1, wtc,
               bt1, wt2, bt2, out):
    ctxv = ctx[...]
    evpv = evp[...]
    p1 = ctxv * evpv
    dot_raw = jnp.dot(p1, s3[...], preferred_element_type=F32)
    n2_raw = jnp.dot(ctxv * ctxv, s3[...], preferred_element_type=F32)
    cnt = jnp.dot(ctxv, e3[...], preferred_element_type=F32)
    elv = jnp.dot(evpv, e4[...], preferred_element_type=F32)
    c = jnp.maximum(cnt, 1.0)
    cos = jnp.clip(dot_raw / (elv * (jnp.sqrt(n2_raw) + 1e-6 * c)),
                   -1.0, 1.0)

    def sig(v):
        return 0.5 + 0.5 * jnp.tanh(0.5 * v)

    h1v = gath[...] + b1[...] + jnp.dot(ea[...], w1b[...],
                                        preferred_element_type=F32)
    h1 = h1v * sig(h1v)
    mv = jnp.dot(h1, w2[...], preferred_element_type=F32) + b2[...]
    m = mv * sig(mv)
    tv = (jnp.dot(m, wt1[...], preferred_element_type=F32)
          + cos * wtc[...] + bt1[...])
    t = tv * sig(tv)
    g = sig(jnp.dot(t, wt2[...], preferred_element_type=F32) + bt2[...])
    out[...] = m * g


def _edge_mlp(gath, ea, ctx, evp, s3, e3, e4, w1b, b1, w2, b2, wt1, wtc, bt1,
              wt2, bt2):
    be = 4000
    grid = (_E // be,)
    wspec = pl.BlockSpec((_D, _D), lambda i: (0, 0))
    bspec = pl.BlockSpec((1, _D), lambda i: (0, 0))
    sspec = pl.BlockSpec((16, _D), lambda i: (0, 0))
    return pl.pallas_call(
        _edge_body,
        grid=grid,
        in_specs=[
            pl.BlockSpec((be, _D), lambda i: (i, 0)),
            pl.BlockSpec((be, 16), lambda i: (i, 0)),
            pl.BlockSpec((be, 16), lambda i: (i, 0)),
            pl.BlockSpec((be, 16), lambda i: (i, 0)),
            sspec, sspec, sspec, sspec,
            bspec, wspec, bspec, wspec, bspec, bspec, wspec, bspec,
        ],
        out_specs=pl.BlockSpec((be, _D), lambda i: (i, 0)),
        out_shape=jax.ShapeDtypeStruct((_E, _D), F32),
    )(gath, ea, ctx, evp, s3, e3, e4, w1b, b1, w2, b2, wt1, wtc, bt1, wt2,
      bt2)


def _sc_scatter_body(mp, col2d, znaux, out, idx0, idx1, mbuf0, mbuf1,
                     aggr_sh, lsem0, lsem1, ssem0, ssem1):
    cid = lax.axis_index("c")
    sid = lax.axis_index("s")
    wid = sid * _NC + cid
    base = wid * _EPW
    idxs = (idx0, idx1)
    mbufs = (mbuf0, mbuf1)
    lsems, ssems = (lsem0, lsem1), (ssem0, ssem1)

    pltpu.sync_copy(znaux.at[pl.ds(sid * _NRT, _NRT)],
                    aggr_sh.at[pl.ds(sid * _NRT, _NRT)])
    plsc.subcore_barrier()

    def fire_in(g, b):
        pltpu.async_copy(col2d.at[wid, g], idxs[b], lsems[b])
        pltpu.async_copy(mp.at[pl.ds(base + g * _G, _G)], mbufs[b],
                         lsems[b])

    def wait_in(b):
        pltpu.make_async_copy(col2d.at[0, 0], idxs[b], lsems[b]).wait()
        pltpu.make_async_copy(mp.at[pl.ds(0, _G)], mbufs[b], lsems[b]).wait()

    def consume(g, b):
        wait_in(b)
        pltpu.async_copy(mbufs[b], aggr_sh.at[idxs[b]], ssems[b], add=True)

    def wait_scat(b):
        pltpu.make_async_copy(mp.at[pl.ds(0, _G)], mbufs[b], ssems[b]).wait()

    fire_in(0, 0)
    fire_in(1, 1)
    consume(0, 0)

    def body(g, b, prefetch=True):
        wait_scat(1 - b)
        if prefetch:
            fire_in(g + 1, 1 - b)
        consume(g, b)

    body(1, 1)

    def pair(p, _):
        body(2 * p, 0)
        body(2 * p + 1, 1)
        return 0

    lax.fori_loop(1, (_NG1 - 1) // 2, pair, 0)
    body(_NG1 - 1, 0, prefetch=False)
    wait_scat(0)
    plsc.subcore_barrier()

    for q in range(_NRT // _ZR):
        r0 = sid * _NRT + q * _ZR
        pltpu.sync_copy(aggr_sh.at[pl.ds(r0, _ZR)],
                        out.at[cid, pl.ds(r0, _ZR)])


def _sc_scatter(mp, col2d):
    mesh = plsc.VectorSubcoreMesh(core_axis_name="c", subcore_axis_name="s")
    f = functools.partial(
        pl.kernel,
        out_type=jax.ShapeDtypeStruct((_NC, _NP, _D), F32),
        mesh=mesh,
        scratch_types=[
            pltpu.VMEM((_G,), jnp.int32),
            pltpu.VMEM((_G,), jnp.int32),
            pltpu.VMEM((_G, _D), F32),
            pltpu.VMEM((_G, _D), F32),
            pltpu.VMEM_SHARED((_NP, _D), F32),
            pltpu.SemaphoreType.DMA,
            pltpu.SemaphoreType.DMA,
            pltpu.SemaphoreType.DMA,
            pltpu.SemaphoreType.DMA,
        ],
    )(_sc_scatter_body)
    return f(mp, col2d, jnp.zeros((_NP, _D), F32))


def _node_body(*refs):
    x_ref = refs[0]
    nparts = len(refs) - 7
    parts = refs[1:1 + nparts]
    wna, wnb, b1, w2, b2, out = refs[1 + nparts:]
    xv = x_ref[...]
    aggr = parts[0][0]
    for pr in parts[1:]:
        aggr = aggr + pr[0]
    h = _swish(jnp.dot(xv, wna[...], preferred_element_type=F32)
               + jnp.dot(aggr, wnb[...], preferred_element_type=F32)
               + b1[...])
    out[...] = xv + _swish(jnp.dot(h, w2[...], preferred_element_type=F32)
                           + b2[...])


def _node_mlp(x, aggrs, wna, wnb, b1, w2, b2):
    bn = 2000
    grid = (_N // bn,)
    wspec = pl.BlockSpec((_D, _D), lambda i: (0, 0))
    bspec = pl.BlockSpec((1, _D), lambda i: (0, 0))
    pspecs = []
    pargs = []
    for a in aggrs:
        pspecs.append(pl.BlockSpec((1, bn, _D), lambda i: (0, i, 0)))
        pspecs.append(pl.BlockSpec((1, bn, _D), lambda i: (1, i, 0)))
        pargs.extend([a, a])
    return pl.pallas_call(
        _node_body,
        grid=grid,
        in_specs=[pl.BlockSpec((bn, _D), lambda i: (i, 0))] + pspecs
        + [wspec, wspec, bspec, wspec, bspec],
        out_specs=pl.BlockSpec((bn, _D), lambda i: (i, 0)),
        out_shape=jax.ShapeDtypeStruct((_N, _D), F32),
    )(x, *pargs, wna, wnb, b1, w2, b2)


def kernel(x, edge_index, edge_attr, edge_vec, edge_length,
           We1, be1, We2, be2, Wn1, bn1, Wn2, bn2, Wt1, bt1, Wt2, bt2):
    D = _D
    R = edge_attr.shape[1]
    row = edge_index[0]
    col = edge_index[1]

    We1a, We1b, We1c = We1[:D], We1[D:D + R], We1[D + R:]
    el2d = edge_length[:, None]
    evp = jnp.concatenate(
        [edge_vec,
         jnp.ones((_E, 1), F32),
         el2d,
         jnp.zeros((_E, 11), F32)], axis=1)
    row2d = row.reshape(_NW, _NG1, _G)
    col2d = col.reshape(_NW, _NG1, _G)
    lane16 = jnp.arange(16, dtype=jnp.int32)[:, None]
    s3 = jnp.tile((lane16 < 3).astype(F32), (1, _D))
    e3 = jnp.tile((lane16 == 3).astype(F32), (1, _D))
    e4 = jnp.tile((lane16 == 4).astype(F32), (1, _D))

    xa, xc = _precomp(x, We1a, We1c)
    naux2 = _sc_vecstats(evp, col2d)
    naux = _combine(naux2)
    gath, ctx = _sc_gather(xa, xc, naux, row2d, col2d)
    mp = _edge_mlp(gath, edge_attr, ctx, evp, s3, e3, e4,
                   We1b, be1[None, :], We2, be2[None, :],
                   Wt1[:D], Wt1[D:D + 1], bt1[None, :], Wt2, bt2[None, :])
    aggrs = [_sc_scatter(mp, col2d)]
    return _node_mlp(x, aggrs, Wn1[:D], Wn1[D:], bn1[None, :], Wn2,
                     bn2[None, :])


# speedup vs baseline: 4.5659x; 1.0212x over previous
import functools

import jax
import jax.numpy as jnp
from jax import lax
from jax.experimental import pallas as pl
from jax.experimental.pallas import tpu as pltpu
from jax.experimental.pallas import tpu_sc as plsc

F32 = jnp.float32

_N = 10000
_E = 320000
_D = 128

_NC = 2
_NS = 16
_NW = _NC * _NS
_G = 80
_EPW = _E // _NW
_NG1 = _EPW // _G
_EPH = _E // _NC
_NP = 10240
_NRT = _NP // _NS
_ZR = 128
_CK = 5
_GPC = _NG1 // _CK
_EPC = _E // _CK


def _swish(v):
    return v * jax.nn.sigmoid(v)


def _precomp_body(x_ref, wa_ref, wc_ref, xa_ref, xc_ref):
    xv = x_ref[...]
    xa_ref[...] = jnp.dot(xv, wa_ref[...], preferred_element_type=F32)
    xc_ref[...] = jnp.dot(xv, wc_ref[...], preferred_element_type=F32)


def _precomp(x, wa, wc):
    bn = 2000
    grid = (_N // bn,)
    return pl.pallas_call(
        _precomp_body,
        grid=grid,
        in_specs=[
            pl.BlockSpec((bn, _D), lambda i: (i, 0)),
            pl.BlockSpec((_D, _D), lambda i: (0, 0)),
            pl.BlockSpec((_D, _D), lambda i: (0, 0)),
        ],
        out_specs=[
            pl.BlockSpec((bn, _D), lambda i: (i, 0)),
            pl.BlockSpec((bn, _D), lambda i: (i, 0)),
        ],
        out_shape=[
            jax.ShapeDtypeStruct((_N, _D), F32),
            jax.ShapeDtypeStruct((_N, _D), F32),
        ],
    )(x, wa, wc)


def _evp_body(ev, el, out):
    bn = ev.shape[0]
    out[...] = jnp.concatenate(
        [ev[...], jnp.ones((bn, 1), F32), el[...],
         jnp.zeros((bn, 11), F32)], axis=1)


def _build_evp(edge_vec, edge_length):
    bn = 20000
    grid = (_E // bn,)
    return pl.pallas_call(
        _evp_body,
        grid=grid,
        in_specs=[
            pl.BlockSpec((bn, 3), lambda i: (i, 0)),
            pl.BlockSpec((bn, 1), lambda i: (i, 0)),
        ],
        out_specs=pl.BlockSpec((bn, 16), lambda i: (i, 0)),
        out_shape=jax.ShapeDtypeStruct((_E, 16), F32),
    )(edge_vec, edge_length)


def _sc_vecstats_body(evp, col2d, znaux, out, idx0, idx1, ebuf0, ebuf1,
                      wbuf0, wbuf1, naux_sh, esem0, esem1, ssem0, ssem1):
    cid = lax.axis_index("c")
    sid = lax.axis_index("s")
    base = cid * _EPH + sid * _EPW
    chunk = base // _EPW
    idxs = (idx0, idx1)
    ebufs, wbufs = (ebuf0, ebuf1), (wbuf0, wbuf1)
    esems, ssems = (esem0, esem1), (ssem0, ssem1)

    def fire_in(g, b):
        pltpu.async_copy(col2d.at[chunk, g], idxs[b], esems[b])
        pltpu.async_copy(evp.at[pl.ds(base + g * _G, _G)], ebufs[b], esems[b])

    def wait_in(b):
        pltpu.make_async_copy(col2d.at[0, 0], idxs[b], esems[b]).wait()
        pltpu.make_async_copy(evp.at[pl.ds(0, _G)], ebufs[b], esems[b]).wait()

    def wait_scat(b):
        pltpu.make_async_copy(znaux.at[pl.ds(0, _G)], wbufs[b],
                              ssems[b]).wait()

    def zw(r, _):
        for k in range(_D // 16):
            wbuf0[r, pl.ds(k * 16, 16)] = jnp.zeros((16,), F32)
            wbuf1[r, pl.ds(k * 16, 16)] = jnp.zeros((16,), F32)
        return 0

    lax.fori_loop(0, _G, zw, 0)

    pltpu.sync_copy(znaux.at[pl.ds(sid * _NRT, _NRT)],
                    naux_sh.at[pl.ds(sid * _NRT, _NRT)])
    plsc.subcore_barrier()

    fire_in(0, 0)
    fire_in(1, 1)

    def consume(g, b):
        wait_in(b)

        def expand(r, _):
            wbufs[b][r, pl.ds(0, 16)] = ebufs[b][r, :]
            return 0

        lax.fori_loop(0, _G, expand, 0)
        pltpu.async_copy(wbufs[b], naux_sh.at[idxs[b]], ssems[b], add=True)

    consume(0, 0)

    def body(g, b, prefetch=True):
        wait_scat(1 - b)
        if prefetch:
            fire_in(g + 1, 1 - b)
        consume(g, b)

    body(1, 1)

    def pair(p, _):
        body(2 * p, 0)
        body(2 * p + 1, 1)
        return 0

    lax.fori_loop(1, (_NG1 - 1) // 2, pair, 0)
    body(_NG1 - 1, 0, prefetch=False)
    wait_scat(0)
    plsc.subcore_barrier()

    pltpu.sync_copy(naux_sh.at[pl.ds(sid * _NRT, _NRT)],
                    out.at[cid, pl.ds(sid * _NRT, _NRT)])


def _sc_vecstats(evp, col2d):
    mesh = plsc.VectorSubcoreMesh(core_axis_name="c", subcore_axis_name="s")
    f = functools.partial(
        pl.kernel,
        out_type=jax.ShapeDtypeStruct((_NC, _NP, _D), F32),
        mesh=mesh,
        scratch_types=[
            pltpu.VMEM((_G,), jnp.int32),
            pltpu.VMEM((_G,), jnp.int32),
            pltpu.VMEM((_G, 16), F32),
            pltpu.VMEM((_G, 16), F32),
            pltpu.VMEM((_G, _D), F32),
            pltpu.VMEM((_G, _D), F32),
            pltpu.VMEM_SHARED((_NP, _D), F32),
            pltpu.SemaphoreType.DMA,
            pltpu.SemaphoreType.DMA,
            pltpu.SemaphoreType.DMA,
            pltpu.SemaphoreType.DMA,
        ],
    )(_sc_vecstats_body)
    return f(evp, col2d, jnp.zeros((_NP, _D), F32))


def _combine_body(a0, a1, out):
    out[...] = a0[0] + a1[0]


def _combine(naux2):
    bn = 2048
    grid = (_NP // bn,)
    return pl.pallas_call(
        _combine_body,
        grid=grid,
        in_specs=[
            pl.BlockSpec((1, bn, _D), lambda i: (0, i, 0)),
            pl.BlockSpec((1, bn, _D), lambda i: (1, i, 0)),
        ],
        out_specs=pl.BlockSpec((bn, _D), lambda i: (i, 0)),
        out_shape=jax.ShapeDtypeStruct((_NP, _D), F32),
    )(naux2, naux2)


def _sc_gather_body(xa, xc, naux, row2d, col2d, gath, ctx,
                    ridx0, ridx1, cidx0, cidx1, abuf0, abuf1, bbuf0, bbuf1,
                    sbuf0, sbuf1, cbuf0, cbuf1, isem0, isem1, gsem0, gsem1,
                    osem0, osem1):
    cid = lax.axis_index("c")
    sid = lax.axis_index("s")
    wid = sid * _NC + cid
    base = wid * _EPW
    ridxs, cidxs = (ridx0, ridx1), (cidx0, cidx1)
    abufs, bbufs, sbufs, cbufs = ((abuf0, abuf1), (bbuf0, bbuf1),
                                  (sbuf0, sbuf1), (cbuf0, cbuf1))
    isems, gsems, osems = (isem0, isem1), (gsem0, gsem1), (osem0, osem1)

    def fire_idx(g, b):
        pltpu.async_copy(row2d.at[wid, g], ridxs[b], isems[b])
        pltpu.async_copy(col2d.at[wid, g], cidxs[b], isems[b])

    def wait_idx(b):
        pltpu.make_async_copy(row2d.at[0, 0], ridxs[b], isems[b]).wait()
        pltpu.make_async_copy(row2d.at[0, 0], cidxs[b], isems[b]).wait()

    def fire_gathers(b):
        pltpu.async_copy(xa.at[ridxs[b]], abufs[b], gsems[b])
        pltpu.async_copy(xc.at[cidxs[b]], bbufs[b], gsems[b])
        pltpu.async_copy(naux.at[cidxs[b]], sbufs[b], gsems[b])

    def wait_gathers(b):
        for _ in range(3):
            pltpu.make_async_copy(xa.at[pl.ds(0, _G)], abufs[b],
                                  gsems[b]).wait()

    def consume(g, b):
        off = base + g * _G
        wait_gathers(b)

        def add_row(r, _):
            for k in range(_D // 16):
                s = pl.ds(k * 16, 16)
                abufs[b][r, s] = abufs[b][r, s] + bbufs[b][r, s]
            return 0

        lax.fori_loop(0, _G, add_row, 0)

        def compact(r, _):
            cbufs[b][r, :] = sbufs[b][r, pl.ds(0, 16)]
            return 0

        lax.fori_loop(0, _G, compact, 0)
        pltpu.async_copy(abufs[b], gath.at[pl.ds(off, _G)], osems[b])
        pltpu.async_copy(cbufs[b], ctx.at[pl.ds(off, _G)], osems[b])

    def wait_outs(b):
        pltpu.make_async_copy(xa.at[pl.ds(0, _G)], abufs[b], osems[b]).wait()
        pltpu.make_async_copy(ctx.at[pl.ds(0, _G)], cbufs[b], osems[b]).wait()

    fire_idx(0, 0)
    fire_idx(1, 1)
    wait_idx(0)
    fire_gathers(0)
    wait_idx(1)
    fire_gathers(1)
    consume(0, 0)

    def body(g, b, prefetch=True):
        if prefetch:
            fire_idx(g + 1, 1 - b)
        wait_outs(1 - b)
        if prefetch:
            wait_idx(1 - b)
            fire_gathers(1 - b)
        consume(g, b)

    body(1, 1)

    def pair(p, _):
        body(2 * p, 0)
        body(2 * p + 1, 1)
        return 0

    lax.fori_loop(1, (_NG1 - 1) // 2, pair, 0)
    body(_NG1 - 1, 0, prefetch=False)
    wait_outs(0)


def _sc_gather(xa, xc, naux, row2d, col2d):
    mesh = plsc.VectorSubcoreMesh(core_axis_name="c", subcore_axis_name="s")
    f = functools.partial(
        pl.kernel,
        out_type=[
            jax.ShapeDtypeStruct((_E, _D), F32),
            jax.ShapeDtypeStruct((_E, 16), F32),
        ],
        mesh=mesh,
        scratch_types=[
            pltpu.VMEM((_G,), jnp.int32),
            pltpu.VMEM((_G,), jnp.int32),
            pltpu.VMEM((_G,), jnp.int32),
            pltpu.VMEM((_G,), jnp.int32),
            pltpu.VMEM((_G, _D), F32),
            pltpu.VMEM((_G, _D), F32),
            pltpu.VMEM((_G, _D), F32),
            pltpu.VMEM((_G, _D), F32),
            pltpu.VMEM((_G, _D), F32),
            pltpu.VMEM((_G, _D), F32),
            pltpu.VMEM((_G, 16), F32),
            pltpu.VMEM((_G, 16), F32),
            pltpu.SemaphoreType.DMA,
            pltpu.SemaphoreType.DMA,
            pltpu.SemaphoreType.DMA,
            pltpu.SemaphoreType.DMA,
            pltpu.SemaphoreType.DMA,
            pltpu.SemaphoreType.DMA,
        ],
    )(_sc_gather_body)
    return f(xa, xc, naux, row2d, col2d)


def _edge_body(gath, ea, ctx, evp, s3, e3, e4, w1b, b1, w2, b2, wt1, wtc,
               bt1, wt2, bt2, out):
    ctxv = ctx[...]
    evpv = evp[...]
    p1 = ctxv * evpv
    dot_raw = jnp.dot(p1, s3[...], preferred_element_type=F32)
    n2_raw = jnp.dot(ctxv * ctxv, s3[...], preferred_element_type=F32)
    cnt = jnp.dot(ctxv, e3[...], preferred_element_type=F32)
    elv = jnp.dot(evpv, e4[...], preferred_element_type=F32)
    c = jnp.maximum(cnt, 1.0)
    cos = jnp.clip(dot_raw / (elv * (jnp.sqrt(n2_raw) + 1e-6 * c)),
                   -1.0, 1.0)

    def sig(v):
        return 0.5 + 0.5 * jnp.tanh(0.5 * v)

    h1v = gath[...] + b1[...] + jnp.dot(ea[...], w1b[...],
                                        preferred_element_type=F32)
    h1 = h1v * sig(h1v)
    mv = jnp.dot(h1, w2[...], preferred_element_type=F32) + b2[...]
    m = mv * sig(mv)
    tv = (jnp.dot(m, wt1[...], preferred_element_type=F32)
          + cos * wtc[...] + bt1[...])
    t = tv * sig(tv)
    g = sig(jnp.dot(t, wt2[...], preferred_element_type=F32) + bt2[...])
    out[...] = m * g


def _edge_mlp(gath, ea, ctx, evp, s3, e3, e4, w1b, b1, w2, b2, wt1, wtc, bt1,
              wt2, bt2):
    be = 8000
    grid = (_E // be,)
    wspec = pl.BlockSpec((_D, _D), lambda i: (0, 0))
    bspec = pl.BlockSpec((1, _D), lambda i: (0, 0))
    sspec = pl.BlockSpec((16, _D), lambda i: (0, 0))
    return pl.pallas_call(
        _edge_body,
        grid=grid,
        in_specs=[
            pl.BlockSpec((be, _D), lambda i: (i, 0)),
            pl.BlockSpec((be, 16), lambda i: (i, 0)),
            pl.BlockSpec((be, 16), lambda i: (i, 0)),
            pl.BlockSpec((be, 16), lambda i: (i, 0)),
            sspec, sspec, sspec, sspec,
            bspec, wspec, bspec, wspec, bspec, bspec, wspec, bspec,
        ],
        out_specs=pl.BlockSpec((be, _D), lambda i: (i, 0)),
        out_shape=jax.ShapeDtypeStruct((_E, _D), F32),
    )(gath, ea, ctx, evp, s3, e3, e4, w1b, b1, w2, b2, wt1, wtc, bt1, wt2,
      bt2)


def _sc_scatter_body(mp, col2d, znaux, out, idx0, idx1, mbuf0, mbuf1,
                     aggr_sh, lsem0, lsem1, ssem0, ssem1):
    cid = lax.axis_index("c")
    sid = lax.axis_index("s")
    wid = sid * _NC + cid
    base = wid * _EPW
    idxs = (idx0, idx1)
    mbufs = (mbuf0, mbuf1)
    lsems, ssems = (lsem0, lsem1), (ssem0, ssem1)

    pltpu.sync_copy(znaux.at[pl.ds(sid * _NRT, _NRT)],
                    aggr_sh.at[pl.ds(sid * _NRT, _NRT)])
    plsc.subcore_barrier()

    def fire_in(g, b):
        pltpu.async_copy(col2d.at[wid, g], idxs[b], lsems[b])
        pltpu.async_copy(mp.at[pl.ds(base + g * _G, _G)], mbufs[b],
                         lsems[b])

    def wait_in(b):
        pltpu.make_async_copy(col2d.at[0, 0], idxs[b], lsems[b]).wait()
        pltpu.make_async_copy(mp.at[pl.ds(0, _G)], mbufs[b], lsems[b]).wait()

    def consume(g, b):
        wait_in(b)
        pltpu.async_copy(mbufs[b], aggr_sh.at[idxs[b]], ssems[b], add=True)

    def wait_scat(b):
        pltpu.make_async_copy(mp.at[pl.ds(0, _G)], mbufs[b], ssems[b]).wait()

    fire_in(0, 0)
    fire_in(1, 1)
    consume(0, 0)

    def body(g, b, prefetch=True):
        wait_scat(1 - b)
        if prefetch:
            fire_in(g + 1, 1 - b)
        consume(g, b)

    body(1, 1)

    def pair(p, _):
        body(2 * p, 0)
        body(2 * p + 1, 1)
        return 0

    lax.fori_loop(1, (_NG1 - 1) // 2, pair, 0)
    body(_NG1 - 1, 0, prefetch=False)
    wait_scat(0)
    plsc.subcore_barrier()

    for q in range(_NRT // _ZR):
        r0 = sid * _NRT + q * _ZR
        pltpu.sync_copy(aggr_sh.at[pl.ds(r0, _ZR)],
                        out.at[cid, pl.ds(r0, _ZR)])


def _sc_scatter(mp, col2d):
    mesh = plsc.VectorSubcoreMesh(core_axis_name="c", subcore_axis_name="s")
    f = functools.partial(
        pl.kernel,
        out_type=jax.ShapeDtypeStruct((_NC, _NP, _D), F32),
        mesh=mesh,
        scratch_types=[
            pltpu.VMEM((_G,), jnp.int32),
            pltpu.VMEM((_G,), jnp.int32),
            pltpu.VMEM((_G, _D), F32),
            pltpu.VMEM((_G, _D), F32),
            pltpu.VMEM_SHARED((_NP, _D), F32),
            pltpu.SemaphoreType.DMA,
            pltpu.SemaphoreType.DMA,
            pltpu.SemaphoreType.DMA,
            pltpu.SemaphoreType.DMA,
        ],
    )(_sc_scatter_body)
    return f(mp, col2d, jnp.zeros((_NP, _D), F32))


def _node_body(*refs):
    x_ref = refs[0]
    nparts = len(refs) - 7
    parts = refs[1:1 + nparts]
    wna, wnb, b1, w2, b2, out = refs[1 + nparts:]
    xv = x_ref[...]
    aggr = parts[0][0]
    for pr in parts[1:]:
        aggr = aggr + pr[0]
    h = _swish(jnp.dot(xv, wna[...], preferred_element_type=F32)
               + jnp.dot(aggr, wnb[...], preferred_element_type=F32)
               + b1[...])
    out[...] = xv + _swish(jnp.dot(h, w2[...], preferred_element_type=F32)
                           + b2[...])


def _node_mlp(x, aggrs, wna, wnb, b1, w2, b2):
    bn = 2000
    grid = (_N // bn,)
    wspec = pl.BlockSpec((_D, _D), lambda i: (0, 0))
    bspec = pl.BlockSpec((1, _D), lambda i: (0, 0))
    pspecs = []
    pargs = []
    for a in aggrs:
        pspecs.append(pl.BlockSpec((1, bn, _D), lambda i: (0, i, 0)))
        pspecs.append(pl.BlockSpec((1, bn, _D), lambda i: (1, i, 0)))
        pargs.extend([a, a])
    return pl.pallas_call(
        _node_body,
        grid=grid,
        in_specs=[pl.BlockSpec((bn, _D), lambda i: (i, 0))] + pspecs
        + [wspec, wspec, bspec, wspec, bspec],
        out_specs=pl.BlockSpec((bn, _D), lambda i: (i, 0)),
        out_shape=jax.ShapeDtypeStruct((_N, _D), F32),
    )(x, *pargs, wna, wnb, b1, w2, b2)


def kernel(x, edge_index, edge_attr, edge_vec, edge_length,
           We1, be1, We2, be2, Wn1, bn1, Wn2, bn2, Wt1, bt1, Wt2, bt2):
    D = _D
    R = edge_attr.shape[1]
    row = edge_index[0]
    col = edge_index[1]

    We1a, We1b, We1c = We1[:D], We1[D:D + R], We1[D + R:]
    el2d = edge_length[:, None]
    evp = jnp.concatenate(
        [edge_vec,
         jnp.ones((_E, 1), F32),
         el2d,
         jnp.zeros((_E, 11), F32)], axis=1)
    row2d = row.reshape(_NW, _NG1, _G)
    col2d = col.reshape(_NW, _NG1, _G)
    lane16 = jnp.arange(16, dtype=jnp.int32)[:, None]
    s3 = jnp.tile((lane16 < 3).astype(F32), (1, _D))
    e3 = jnp.tile((lane16 == 3).astype(F32), (1, _D))
    e4 = jnp.tile((lane16 == 4).astype(F32), (1, _D))

    xa, xc = _precomp(x, We1a, We1c)
    naux2 = _sc_vecstats(evp, col2d)
    naux = _combine(naux2)
    gath, ctx = _sc_gather(xa, xc, naux, row2d, col2d)
    mp = _edge_mlp(gath, edge_attr, ctx, evp, s3, e3, e4,
                   We1b, be1[None, :], We2, be2[None, :],
                   Wt1[:D], Wt1[D:D + 1], bt1[None, :], Wt2, bt2[None, :])
    aggrs = [_sc_scatter(mp, col2d)]
    return _node_mlp(x, aggrs, Wn1[:D], Wn1[D:], bn1[None, :], Wn2,
                     bn2[None, :])
